# Initial kernel scaffold; baseline (speedup 1.0000x reference)
import jax
import jax.numpy as jnp
from jax.experimental import pallas as pl


def kernel(img):
    raise NotImplementedError("write your pallas kernel here")


# speedup vs baseline: 131.4244x; 131.4244x over previous
import functools
import math

import jax
import jax.numpy as jnp
from jax.experimental import pallas as pl
from jax.experimental.pallas import tpu as pltpu

_ORIENTATIONS = 10
_PPC = 8
_CPB = 2
_MAX_ANGLE = math.pi
_EPS = 1e-5

_H = 512
_W = 512
_HC = _H // _PPC
_WC = _W // _PPC
_HN = _HC - _CPB + 1
_WN = _WC - _CPB + 1


def _hog_body(img_ref, pool_ref, out_ref):
    p = img_ref[0].astype(jnp.bfloat16).astype(jnp.float32)

    def tap(dy, dx):
        return p[dy:dy + _H, dx:dx + _W]

    gx = ((((tap(0, 0) - tap(0, 2)) + 2.0 * tap(1, 0))
           - 2.0 * tap(1, 2)) + tap(2, 0)) - tap(2, 2)
    gy = ((((tap(0, 0) + 2.0 * tap(0, 1)) + tap(0, 2))
           - tap(2, 0)) - 2.0 * tap(2, 1)) - tap(2, 2)

    mag = jnp.sqrt(gx * gx + gy * gy)
    t = jnp.arctan2(gx, gy) / _MAX_ANGLE * _ORIENTATIONS
    fo = jnp.mod(jnp.floor(t).astype(jnp.int32), _ORIENTATIONS)
    ce = jnp.mod(jnp.ceil(t).astype(jnp.int32), _ORIENTATIONS)
    one_minus = 1.0 - mag

    rows = []
    for o in range(_ORIENTATIONS):
        contrib = (jnp.where(fo == o, mag, 0.0)
                   + jnp.where(ce == o, one_minus, 0.0))
        rows.append(contrib.reshape(_HC, _PPC, _W).sum(axis=1))
    rp = jnp.concatenate(rows, axis=0)

    pooled = jax.lax.dot(rp, pool_ref[...],
                         precision=jax.lax.Precision.HIGHEST)
    h = pooled.reshape(_ORIENTATIONS, _HC, _WC) * (1.0 / (_PPC * _PPC))

    h00 = h[:, 0:_HN, 0:_WN]
    h01 = h[:, 0:_HN, 1:_WC]
    h10 = h[:, 1:_HC, 0:_WN]
    h11 = h[:, 1:_HC, 1:_WC]
    blk = h00 + h01 + h10 + h11
    inv = 1.0 / jnp.sqrt(blk * blk + _EPS * _EPS)
    out_ref[0, 0, :, 0:_HN, 0:_WN] = h00 * inv
    out_ref[0, 1, :, 0:_HN, 0:_WN] = h01 * inv
    out_ref[0, 2, :, 0:_HN, 0:_WN] = h10 * inv
    out_ref[0, 3, :, 0:_HN, 0:_WN] = h11 * inv


@jax.jit
def kernel(img):
    n = img.shape[0]
    x = jnp.pad(img[:, 0, :, :], ((0, 0), (1, 1), (1, 1)))
    pool = (jnp.arange(_W)[:, None] // _PPC
            == jnp.arange(_WC)[None, :]).astype(jnp.float32)

    out = pl.pallas_call(
        _hog_body,
        grid=(n,),
        in_specs=[
            pl.BlockSpec((1, _H + 2, _W + 2), lambda i: (i, 0, 0)),
            pl.BlockSpec((_W, _WC), lambda i: (0, 0)),
        ],
        out_specs=pl.BlockSpec((1, _CPB * _CPB, _ORIENTATIONS, _HC, _WC),
                               lambda i: (i, 0, 0, 0, 0)),
        out_shape=jax.ShapeDtypeStruct(
            (n, _CPB * _CPB, _ORIENTATIONS, _HC, _WC), jnp.float32),
        compiler_params=pltpu.CompilerParams(
            dimension_semantics=("arbitrary",)),
    )(x, pool)

    y = out[:, :, :, :_HN, :_WN]
    y = jnp.transpose(y, (0, 2, 3, 4, 1))
    return y.reshape(n, _ORIENTATIONS * _HN * _WN * _CPB * _CPB)


# speedup vs baseline: 164.2785x; 1.2500x over previous
import math

import jax
import jax.numpy as jnp
from jax.experimental import pallas as pl
from jax.experimental.pallas import tpu as pltpu

_ORIENTATIONS = 10
_PPC = 8
_CPB = 2
_MAX_ANGLE = math.pi
_EPS = 1e-5

_H = 512
_W = 512
_HC = _H // _PPC
_WC = _W // _PPC
_HN = _HC - _CPB + 1
_WN = _WC - _CPB + 1
_FW = _WN * _CPB * _CPB


def _hog_body(img_ref, pool_ref, exp_ref, out_ref, pad_ref):
    @pl.when(pl.program_id(0) == 0)
    def _():
        pad_ref[0:1, :] = jnp.zeros((1, _W + 2), jnp.float32)
        pad_ref[_H + 1:_H + 2, :] = jnp.zeros((1, _W + 2), jnp.float32)
        pad_ref[:, 0:1] = jnp.zeros((_H + 2, 1), jnp.float32)
        pad_ref[:, _W + 1:_W + 2] = jnp.zeros((_H + 2, 1), jnp.float32)

    pad_ref[1:_H + 1, 1:_W + 1] = (
        img_ref[0, 0].astype(jnp.bfloat16).astype(jnp.float32))
    slab_rows = [[] for _ in range(_ORIENTATIONS)]
    n_slabs = _H // 64
    for s in range(n_slabs):
        p = pad_ref[64 * s:64 * s + 66, :]

        def tap(dy, dx):
            return p[dy:dy + 64, dx:dx + _W]

        gx = ((((tap(0, 0) - tap(0, 2)) + 2.0 * tap(1, 0))
               - 2.0 * tap(1, 2)) + tap(2, 0)) - tap(2, 2)
        gy = ((((tap(0, 0) + 2.0 * tap(0, 1)) + tap(0, 2))
               - tap(2, 0)) - 2.0 * tap(2, 1)) - tap(2, 2)

        mag = jnp.sqrt(gx * gx + gy * gy)
        t = jnp.arctan2(gx, gy) / _MAX_ANGLE * _ORIENTATIONS
        f = jnp.floor(t)
        is_int = t == f
        fi = f.astype(jnp.int32)
        fo = jnp.where(fi < 0, fi + _ORIENTATIONS, fi)
        fo = jnp.where(fo == _ORIENTATIONS, 0, fo)
        one_minus = 1.0 - mag
        a = jnp.where(is_int, mag + one_minus, mag)
        b = jnp.where(is_int, 0.0, one_minus)

        zero = jnp.zeros_like(mag)
        prev = fo == (_ORIENTATIONS - 1)
        for o in range(_ORIENTATIONS):
            m = fo == o
            contrib = jnp.where(m, a, zero) + jnp.where(prev, b, zero)
            slab_rows[o].append(contrib.reshape(8, _PPC, _W).sum(axis=1))
            prev = m
    rp = jnp.concatenate(
        [r for o in range(_ORIENTATIONS) for r in slab_rows[o]],
        axis=0)

    pooled = jax.lax.dot(rp, pool_ref[...],
                         precision=jax.lax.Precision.HIGHEST)
    h = pooled.reshape(_ORIENTATIONS, _HC, _WC) * (1.0 / (_PPC * _PPC))

    h00 = h[:, 0:_HN, 0:_WN]
    h01 = h[:, 0:_HN, 1:_WC]
    h10 = h[:, 1:_HC, 0:_WN]
    h11 = h[:, 1:_HC, 1:_WC]
    blk = h00 + h01 + h10 + h11
    inv = 1.0 / jnp.sqrt(blk * blk + _EPS * _EPS)
    y00 = h00 * inv
    y01 = h01 * inv
    y10 = h10 * inv
    y11 = h11 * inv
    ys = [y00, y01, y10, y11]
    his = [y.astype(jnp.bfloat16).astype(jnp.float32) for y in ys]
    los = [y - hi for y, hi in zip(ys, his)]
    for o in range(_ORIENTATIONS):
        acc = jax.lax.dot(his[0][o], exp_ref[0])
        for q in range(1, _CPB * _CPB):
            acc += jax.lax.dot(his[q][o], exp_ref[q])
        for q in range(_CPB * _CPB):
            acc += jax.lax.dot(los[q][o], exp_ref[q])
        out_ref[0, o] = acc


@jax.jit
def kernel(img):
    n = img.shape[0]
    pool = (jnp.arange(_W)[:, None] // _PPC
            == jnp.arange(_WC)[None, :]).astype(jnp.float32)
    exp = (jnp.arange(_FW)[None, None, :]
           == 4 * jnp.arange(_WN)[None, :, None]
           + jnp.arange(_CPB * _CPB)[:, None, None]).astype(jnp.float32)

    out = pl.pallas_call(
        _hog_body,
        grid=(n,),
        in_specs=[
            pl.BlockSpec((1, 1, _H, _W), lambda i: (i, 0, 0, 0)),
            pl.BlockSpec((_W, _WC), lambda i: (0, 0)),
            pl.BlockSpec((_CPB * _CPB, _WN, _FW), lambda i: (0, 0, 0)),
        ],
        out_specs=pl.BlockSpec((1, _ORIENTATIONS, _HN, _FW),
                               lambda i: (i, 0, 0, 0)),
        out_shape=jax.ShapeDtypeStruct(
            (n, _ORIENTATIONS, _HN, _FW), jnp.float32),
        scratch_shapes=[pltpu.VMEM((_H + 2, _W + 2), jnp.float32)],
        compiler_params=pltpu.CompilerParams(
            dimension_semantics=("arbitrary",)),
    )(img, pool, exp)

    return out.reshape(n, _ORIENTATIONS * _HN * _FW)
